# baseline (speedup 1.0000x reference)
import functools

import jax
import jax.numpy as jnp
from jax import lax
from jax.experimental import pallas as pl
from jax.experimental.pallas import tpu as pltpu
from jax.experimental.pallas import tpu_sc as plsc

_SRC = 50
_V = 64
_B = 16384
_F = 100
_OF = _F - 1 + _V
_NC = 2
_NS = 16
_NW = _NC * _NS
_CW = _B // _NW
_C = 128
_NCHUNK = _CW // _C
_L = 16
_G = _C // _L


def _sc_body(xt_hbm, eps_hbm, outt_hbm, x_v, o_v, eps_v, in_sems, out_sems):
    wid = lax.axis_index("s") * _NC + lax.axis_index("c")

    zero = jnp.zeros((_L,), jnp.float32)
    lane = lax.iota(jnp.int32, _L)

    def _in_copy(ci, b):
        base = wid * _CW + ci * _C
        return pltpu.make_async_copy(
            xt_hbm.at[:, pl.ds(base, _C)], x_v.at[b], in_sems.at[b])


    def _out_copy(ci, b):
        base = wid * _CW + ci * _C
        return pltpu.make_async_copy(
            o_v.at[b], outt_hbm.at[:, pl.ds(base, _C)], out_sems.at[b])

    def _compute(ci, b):
        return
        xb = x_v.at[ci]
        ob = o_v.at[b]

        def _left(r, _):
            for g in range(_G):
                ob[r, pl.ds(g * _L, _L)] = xb[r, pl.ds(g * _L, _L)]
            return 0

        lax.fori_loop(0, _SRC, _left, 0)

        def _right(r, _):
            for g in range(_G):
                ob[r + _V - 1, pl.ds(g * _L, _L)] = xb[r, pl.ds(g * _L, _L)]
            return 0

        lax.fori_loop(_SRC + 1, _F, _right, 0)

        def _zero_row(r, _):
            for g in range(_G):
                ob[r, pl.ds(g * _L, _L)] = zero
            return 0

        lax.fori_loop(_SRC, _SRC + _V, _zero_row, 0)

        for g in range(_G):
            cols = lane + g * _L
            codes = xb[_SRC, pl.ds(g * _L, _L)].astype(jnp.int32)
            mask = (codes >= 0) & (codes < _V)
            plsc.store_scatter(ob, [codes + _SRC, cols], eps_vec, mask=mask)

    for ci in range(_NCHUNK):
        _in_copy(ci, ci).start()
    pltpu.sync_copy(eps_hbm, eps_v)
    eps_vec = eps_v[...]
    for ci in range(_NCHUNK):
        b = ci % 2
        _in_copy(ci, ci).wait()
        if ci >= 2:
            _out_copy(ci - 2, b).wait()
        _compute(ci, b)
        _out_copy(ci, b).start()
    _out_copy(_NCHUNK - 2, _NCHUNK % 2).wait()
    _out_copy(_NCHUNK - 1, (_NCHUNK - 1) % 2).wait()


def kernel(x, eps):
    xt = x.T
    eps_r = jnp.broadcast_to(jnp.reshape(eps, (1,)), (_L,))
    mesh = plsc.VectorSubcoreMesh(core_axis_name="c", subcore_axis_name="s")
    k = functools.partial(
        pl.kernel,
        mesh=mesh,
        compiler_params=pltpu.CompilerParams(
            use_tc_tiling_on_sc=True, needs_layout_passes=False),
        out_type=jax.ShapeDtypeStruct((_OF, _B), jnp.float32),
        scratch_types=[
            pltpu.VMEM((_NCHUNK, _F, _C), jnp.float32),
            pltpu.VMEM((2, _OF, _C), jnp.float32),
            pltpu.VMEM((_L,), jnp.float32),
            pltpu.SemaphoreType.DMA((_NCHUNK,)),
            pltpu.SemaphoreType.DMA((2,)),
        ],
    )(_sc_body)
    outt = k(xt, eps_r)
    return outt.T


# speedup vs baseline: 1.0376x; 1.0376x over previous
import functools

import jax
import jax.numpy as jnp
from jax import lax
from jax.experimental import pallas as pl
from jax.experimental.pallas import tpu as pltpu
from jax.experimental.pallas import tpu_sc as plsc

_SRC = 50
_V = 64
_B = 16384
_F = 100
_OF = _F - 1 + _V
_NC = 2
_NS = 16
_NW = _NC * _NS
_CW = _B // _NW
_C = 128
_NCHUNK = _CW // _C
_L = 16
_G = _C // _L


def _sc_body(xt_hbm, eps_hbm, outt_hbm, x_v, o_v, eps_v, in_sems, out_sems):
    wid = lax.axis_index("s") * _NC + lax.axis_index("c")

    zero = jnp.zeros((_L,), jnp.float32)
    lane = lax.iota(jnp.int32, _L)

    def _in_copy(ci, b):
        base = wid * _CW + ci * _C
        return pltpu.make_async_copy(
            xt_hbm.at[:, pl.ds(base, _C)], x_v.at[b], in_sems.at[b])


    def _out_copy(ci, b):
        base = wid * _CW + ci * _C
        return pltpu.make_async_copy(
            o_v.at[b], outt_hbm.at[:, pl.ds(base, _C)], out_sems.at[b])

    def _compute(ci, b):
        xb = x_v.at[ci]
        ob = o_v.at[b]

        def _left(r, _):
            for g in range(_G):
                ob[r, pl.ds(g * _L, _L)] = xb[r, pl.ds(g * _L, _L)]
            return 0

        lax.fori_loop(0, _SRC, _left, 0)

        def _right(r, _):
            for g in range(_G):
                ob[r + _V - 1, pl.ds(g * _L, _L)] = xb[r, pl.ds(g * _L, _L)]
            return 0

        lax.fori_loop(_SRC + 1, _F, _right, 0)

        def _zero_row(r, _):
            for g in range(_G):
                ob[r, pl.ds(g * _L, _L)] = zero
            return 0

        lax.fori_loop(_SRC, _SRC + _V, _zero_row, 0)

        for g in range(_G):
            cols = lane + g * _L
            codes = xb[_SRC, pl.ds(g * _L, _L)].astype(jnp.int32)
            mask = (codes >= 0) & (codes < _V)
            plsc.store_scatter(ob, [codes + _SRC, cols], eps_vec, mask=mask)

    for ci in range(_NCHUNK):
        _in_copy(ci, ci).start()
    pltpu.sync_copy(eps_hbm, eps_v)
    eps_vec = eps_v[...]
    for ci in range(_NCHUNK):
        b = ci % 2
        _in_copy(ci, ci).wait()
        if ci >= 2:
            _out_copy(ci - 2, b).wait()
        _compute(ci, b)
        _out_copy(ci, b).start()
    _out_copy(_NCHUNK - 2, _NCHUNK % 2).wait()
    _out_copy(_NCHUNK - 1, (_NCHUNK - 1) % 2).wait()


def kernel(x, eps):
    xt = x.T
    eps_r = jnp.broadcast_to(jnp.reshape(eps, (1,)), (_L,))
    mesh = plsc.VectorSubcoreMesh(core_axis_name="c", subcore_axis_name="s")
    k = functools.partial(
        pl.kernel,
        mesh=mesh,
        compiler_params=pltpu.CompilerParams(
            use_tc_tiling_on_sc=True, needs_layout_passes=False),
        out_type=jax.ShapeDtypeStruct((_OF, _B), jnp.float32),
        scratch_types=[
            pltpu.VMEM((_NCHUNK, _F, _C), jnp.float32),
            pltpu.VMEM((2, _OF, _C), jnp.float32),
            pltpu.VMEM((_L,), jnp.float32),
            pltpu.SemaphoreType.DMA((_NCHUNK,)),
            pltpu.SemaphoreType.DMA((2,)),
        ],
    )(_sc_body)
    outt = k(xt, eps_r)
    return outt.T
